# baseline (speedup 1.0000x reference)
import functools

import jax
import jax.numpy as jnp
from jax import lax
from jax.experimental import pallas as pl
from jax.experimental.pallas import tpu as pltpu
from jax.experimental.pallas import tpu_sc as plsc

HIDDEN_SIZE = 1024

S_CHUNK = 16
S_NBUF = 4
S_ROWS = 128

P_CHUNK = 32
P_NBUF = 2
P_ROWS = 128

_info = plsc.get_sparse_core_info()
_NC = _info.num_cores
_NS = _info.num_subcores
_NW = _NC * _NS


def _ring_events(n_chunks, nbuf):
    ev = []
    for c in range(min(nbuf, n_chunks)):
        ev.append(("start_in", c))
    for c in range(n_chunks):
        if c >= 1 and c - 1 + nbuf < n_chunks:
            ev.append(("wait_out", c - 1))
            ev.append(("start_in", c - 1 + nbuf))
        ev.append(("wait_in", c))
        ev.append(("start_out", c))
    for c in range(max(0, n_chunks - nbuf), n_chunks):
        ev.append(("wait_out", c))
    return ev


def _interleave(a, b):
    out = []
    na, nb = len(a), len(b)
    ia = ib = 0
    while ia < na or ib < nb:
        if ib * na <= ia * nb and ib < nb:
            out.append(("b", b[ib]))
            ib += 1
        elif ia < na:
            out.append(("a", a[ia]))
            ia += 1
        else:
            out.append(("b", b[ib]))
            ib += 1
    return out


@functools.partial(jax.jit, static_argnames=("seq_length",))
def _position_copy(table, seq_length):
    rows_per_w = seq_length // _NW
    assert S_ROWS + P_ROWS == rows_per_w
    n_s = S_ROWS // S_CHUNK
    n_p = P_ROWS // P_CHUNK
    mesh = plsc.VectorSubcoreMesh(core_axis_name="c", subcore_axis_name="s")

    @functools.partial(
        pl.kernel,
        mesh=mesh,
        out_type=jax.ShapeDtypeStruct((seq_length, HIDDEN_SIZE), jnp.float32),
        scratch_types=(
            [pltpu.VMEM((S_CHUNK, HIDDEN_SIZE), jnp.float32) for _ in range(S_NBUF)]
            + [pltpu.VMEM_SHARED((_NS, P_NBUF * P_CHUNK, HIDDEN_SIZE), jnp.float32)]
            + [pltpu.SemaphoreType.DMA for _ in range(2 * S_NBUF + 2 * P_NBUF)]
        ),
    )
    def copy_kernel(table_hbm, out_hbm, *scratch):
        sbufs = scratch[:S_NBUF]
        shared = scratch[S_NBUF]
        sems = scratch[S_NBUF + 1 :]
        s_isems = sems[:S_NBUF]
        s_osems = sems[S_NBUF : 2 * S_NBUF]
        p_isems = sems[2 * S_NBUF : 2 * S_NBUF + P_NBUF]
        p_osems = sems[2 * S_NBUF + P_NBUF :]

        cid = lax.axis_index("c")
        sid = lax.axis_index("s")
        wid = sid * _NC + cid
        base = wid * rows_per_w
        pbase = base + S_ROWS

        def s_in(c):
            b = c % S_NBUF
            return pltpu.make_async_copy(
                table_hbm.at[pl.ds(base + c * S_CHUNK, S_CHUNK)],
                sbufs[b],
                s_isems[b],
            )

        def s_out(c):
            b = c % S_NBUF
            return pltpu.make_async_copy(
                sbufs[b],
                out_hbm.at[pl.ds(base + c * S_CHUNK, S_CHUNK)],
                s_osems[b],
            )

        def p_in(c):
            b = c % P_NBUF
            return pltpu.make_async_copy(
                table_hbm.at[pl.ds(pbase + c * P_CHUNK, P_CHUNK)],
                shared.at[sid, pl.ds(b * P_CHUNK, P_CHUNK)],
                p_isems[b],
            )

        def p_out(c):
            b = c % P_NBUF
            return pltpu.make_async_copy(
                shared.at[sid, pl.ds(b * P_CHUNK, P_CHUNK)],
                out_hbm.at[pl.ds(pbase + c * P_CHUNK, P_CHUNK)],
                p_osems[b],
            )

        s_ops = {"start_in": lambda c: s_in(c).start(), "wait_in": lambda c: s_in(c).wait(),
                 "start_out": lambda c: s_out(c).start(), "wait_out": lambda c: s_out(c).wait()}
        p_ops = {"start_in": lambda c: p_in(c).start(), "wait_in": lambda c: p_in(c).wait(),
                 "start_out": lambda c: p_out(c).start(), "wait_out": lambda c: p_out(c).wait()}

        for ring, (action, c) in _interleave(
            _ring_events(n_s, S_NBUF), _ring_events(n_p, P_NBUF)
        ):
            (s_ops if ring == "a" else p_ops)[action](c)

    return copy_kernel(table)


def kernel(inputs, table):
    seq_length = inputs.shape[1]
    return _position_copy(table, seq_length)


# speedup vs baseline: 1.0067x; 1.0067x over previous
import functools

import jax
import jax.numpy as jnp
from jax import lax
from jax.experimental import pallas as pl
from jax.experimental.pallas import tpu as pltpu
from jax.experimental.pallas import tpu_sc as plsc

HIDDEN_SIZE = 1024
CHUNK_ROWS = 32
NBUF = 3

_info = plsc.get_sparse_core_info()
_NC = _info.num_cores
_NS = _info.num_subcores
_NW = _NC * _NS


@functools.partial(jax.jit, static_argnames=("seq_length",))
def _position_copy(table, seq_length):
    rows_per_w = seq_length // _NW
    n_chunks = rows_per_w // CHUNK_ROWS
    mesh = plsc.VectorSubcoreMesh(core_axis_name="c", subcore_axis_name="s")

    @functools.partial(
        pl.kernel,
        mesh=mesh,
        out_type=jax.ShapeDtypeStruct((seq_length, HIDDEN_SIZE), jnp.float32),
        scratch_types=(
            [pltpu.VMEM((CHUNK_ROWS, HIDDEN_SIZE), jnp.float32) for _ in range(NBUF)]
            + [pltpu.SemaphoreType.DMA for _ in range(2 * NBUF)]
        ),
    )
    def copy_kernel(table_hbm, out_hbm, *scratch):
        bufs = scratch[:NBUF]
        isems = scratch[NBUF : 2 * NBUF]
        osems = scratch[2 * NBUF :]
        wid = lax.axis_index("s") * _NC + lax.axis_index("c")
        base = wid * rows_per_w

        def in_copy(c):
            b = c % NBUF
            return pltpu.make_async_copy(
                table_hbm.at[pl.ds(base + c * CHUNK_ROWS, CHUNK_ROWS)],
                bufs[b],
                isems[b],
            )

        def out_copy(c):
            b = c % NBUF
            return pltpu.make_async_copy(
                bufs[b],
                out_hbm.at[pl.ds(base + c * CHUNK_ROWS, CHUNK_ROWS)],
                osems[b],
            )

        for c in range(min(NBUF, n_chunks)):
            in_copy(c).start()
        for c in range(n_chunks):
            if c >= 1 and c - 1 + NBUF < n_chunks:
                out_copy(c - 1).wait()
                in_copy(c - 1 + NBUF).start()
            in_copy(c).wait()
            out_copy(c).start()
        for c in range(max(0, n_chunks - NBUF), n_chunks):
            out_copy(c).wait()

    return copy_kernel(table)


def kernel(inputs, table):
    seq_length = inputs.shape[1]
    return _position_copy(table, seq_length)
